# baseline (speedup 1.0000x reference)
import functools

import jax
import jax.numpy as jnp
from jax import lax
from jax.experimental import pallas as pl
from jax.experimental.pallas import tpu as pltpu
from jax.experimental.pallas import tpu_sc as plsc

N_NODES = 10000
N_PAD = 10240
ROWS = N_PAD // 128
E_EDGES = 320000
NUM_CORES = 2
NUM_SUBCORES = 16
NUM_WORKERS = NUM_CORES * NUM_SUBCORES
CHUNK = 128
NCHUNKS = E_EDGES // CHUNK
NCHUNKS_PAD = 2560
CPW_MAX = 80
SLICE = N_PAD // NUM_SUBCORES
LANES = 16
F32 = jnp.float32

_MESH = plsc.VectorSubcoreMesh(core_axis_name="c", subcore_axis_name="s")


def _worker_chunks(wid):
    nch = jnp.where(wid < 31, 80, 20)
    crow = pl.multiple_of(wid * CPW_MAX, 8)
    return crow, nch


def _rsqrt16(d):
    s = jnp.full((LANES,), 1.0, F32)
    for _ in range(10):
        s = jnp.where(d * s * s >= 2.0, s * 0.5, s)
    y = s
    for _ in range(6):
        y = y * (1.5 - 0.5 * d * y * y)
    return y


def _reduce32(stage_v, out_sl):
    def body(i, _):
        sl = pl.ds(i * LANES, LANES)
        acc = jnp.zeros((LANES,), F32)
        for r in range(NUM_WORKERS):
            acc = acc + stage_v[r, sl]
        out_sl[sl] = acc
        return 0
    lax.fori_loop(0, SLICE // LANES, body, 0)


@functools.partial(
    pl.kernel,
    out_type=(
        jax.ShapeDtypeStruct((NUM_WORKERS, N_PAD), F32),
        jax.ShapeDtypeStruct((NUM_WORKERS, N_PAD), F32),
    ),
    mesh=_MESH,
    compiler_params=pltpu.CompilerParams(needs_layout_passes=False),
    scratch_types=(
        pltpu.VMEM((CPW_MAX, CHUNK), jnp.int32),
        pltpu.VMEM((CPW_MAX, CHUNK), jnp.int32),
        pltpu.VMEM((N_PAD,), F32),
        pltpu.VMEM((N_PAD,), F32),
    ),
)
def _deg_kernel(e_h, zeros_h, dop_h, dip_h, sidx_v, didx_v, acc_o_v, acc_i_v):
    c = lax.axis_index("c")
    s = lax.axis_index("s")
    wid = c * NUM_SUBCORES + s
    crow, nch = _worker_chunks(wid)

    pltpu.sync_copy(zeros_h, acc_o_v)
    pltpu.sync_copy(zeros_h, acc_i_v)
    pltpu.sync_copy(e_h.at[0, pl.ds(crow, CPW_MAX)], sidx_v)
    pltpu.sync_copy(e_h.at[1, pl.ds(crow, CPW_MAX)], didx_v)

    ones16 = jnp.ones((LANES,), F32)

    def step(j, _):
        for k in range(CHUNK // LANES):
            sl = pl.ds(k * LANES, LANES)
            plsc.addupdate_scatter(acc_o_v, [sidx_v[j, sl]], ones16)
            plsc.addupdate_scatter(acc_i_v, [didx_v[j, sl]], ones16)
        return 0
    lax.fori_loop(0, nch, step, 0)

    pltpu.sync_copy(acc_o_v, dop_h.at[wid])
    pltpu.sync_copy(acc_i_v, dip_h.at[wid])


def _local_edge_pass(e_h, sidx_v, didx_v, tbl_v, acc_v, crow, nch):
    pltpu.sync_copy(e_h.at[0, pl.ds(crow, CPW_MAX)], sidx_v)
    pltpu.sync_copy(e_h.at[1, pl.ds(crow, CPW_MAX)], didx_v)

    def step(j, _):
        for k in range(CHUNK // LANES):
            sl = pl.ds(k * LANES, LANES)
            vals = plsc.load_gather(tbl_v, [didx_v[j, sl]])
            plsc.addupdate_scatter(acc_v, [sidx_v[j, sl]], vals)
        return 0
    lax.fori_loop(0, nch, step, 0)


@functools.partial(
    pl.kernel,
    out_type=(
        jax.ShapeDtypeStruct((NUM_WORKERS, N_PAD), F32),
        jax.ShapeDtypeStruct((NUM_CORES, N_PAD), F32),
        jax.ShapeDtypeStruct((NUM_CORES, N_PAD), F32),
    ),
    mesh=_MESH,
    compiler_params=pltpu.CompilerParams(needs_layout_passes=False),
    scratch_types=(
        pltpu.VMEM((CPW_MAX, CHUNK), jnp.int32),
        pltpu.VMEM((CPW_MAX, CHUNK), jnp.int32),
        pltpu.VMEM((NUM_WORKERS, SLICE), F32),
        pltpu.VMEM((SLICE,), F32),
        pltpu.VMEM((SLICE,), F32),
        pltpu.VMEM((SLICE,), F32),
        pltpu.VMEM((N_PAD,), F32),
        pltpu.VMEM((N_PAD,), F32),
    ),
)
def _w_kernel(e_h, dop_h, dip_h, zeros_h, wp_h, ns_h, nd_h,
              sidx_v, didx_v, stage_v, red_sl, ns_sl, nd_sl, tbl_v, acc_v):
    c = lax.axis_index("c")
    s = lax.axis_index("s")
    wid = c * NUM_SUBCORES + s
    crow, nch = _worker_chunks(wid)
    base = pl.multiple_of(s * SLICE, 8)

    pltpu.sync_copy(dop_h.at[:, pl.ds(base, SLICE)], stage_v)
    _reduce32(stage_v, red_sl)

    def mk_ns(i, _):
        sl = pl.ds(i * LANES, LANES)
        ns_sl[sl] = _rsqrt16(jnp.maximum(red_sl[sl], 1.0))
        return 0
    lax.fori_loop(0, SLICE // LANES, mk_ns, 0)

    pltpu.sync_copy(dip_h.at[:, pl.ds(base, SLICE)], stage_v)
    _reduce32(stage_v, red_sl)

    def mk_nd(i, _):
        sl = pl.ds(i * LANES, LANES)
        nd_sl[sl] = _rsqrt16(jnp.maximum(red_sl[sl], 1.0))
        return 0
    lax.fori_loop(0, SLICE // LANES, mk_nd, 0)

    pltpu.sync_copy(ns_sl, ns_h.at[c, pl.ds(base, SLICE)])
    pltpu.sync_copy(nd_sl, nd_h.at[c, pl.ds(base, SLICE)])
    plsc.subcore_barrier()

    pltpu.sync_copy(nd_h.at[c], tbl_v)
    pltpu.sync_copy(zeros_h, acc_v)
    _local_edge_pass(e_h, sidx_v, didx_v, tbl_v, acc_v, crow, nch)
    pltpu.sync_copy(acc_v, wp_h.at[wid])


@functools.partial(
    pl.kernel,
    out_type=(
        jax.ShapeDtypeStruct((NUM_WORKERS, N_PAD), F32),
        jax.ShapeDtypeStruct((NUM_CORES, N_PAD), F32),
    ),
    mesh=_MESH,
    compiler_params=pltpu.CompilerParams(needs_layout_passes=False),
    scratch_types=(
        pltpu.VMEM((CPW_MAX, CHUNK), jnp.int32),
        pltpu.VMEM((CPW_MAX, CHUNK), jnp.int32),
        pltpu.VMEM((NUM_WORKERS, SLICE), F32),
        pltpu.VMEM((SLICE,), F32),
        pltpu.VMEM((SLICE,), F32),
        pltpu.VMEM((SLICE,), F32),
        pltpu.VMEM((N_PAD,), F32),
        pltpu.VMEM((N_PAD,), F32),
    ),
)
def _v_kernel(e_h, wp_h, ns_h, nd_h, zeros_h, vp_h, u_h,
              sidx_v, didx_v, stage_v, red_sl, ns_sl, nd_sl, tbl_v, acc_v):
    c = lax.axis_index("c")
    s = lax.axis_index("s")
    wid = c * NUM_SUBCORES + s
    crow, nch = _worker_chunks(wid)
    base = pl.multiple_of(s * SLICE, 8)

    pltpu.sync_copy(wp_h.at[:, pl.ds(base, SLICE)], stage_v)
    _reduce32(stage_v, red_sl)
    pltpu.sync_copy(ns_h.at[c, pl.ds(base, SLICE)], ns_sl)
    pltpu.sync_copy(nd_h.at[c, pl.ds(base, SLICE)], nd_sl)

    def mk_u(i, _):
        sl = pl.ds(i * LANES, LANES)
        red_sl[sl] = nd_sl[sl] * (ns_sl[sl] * red_sl[sl])
        return 0
    lax.fori_loop(0, SLICE // LANES, mk_u, 0)

    pltpu.sync_copy(red_sl, u_h.at[c, pl.ds(base, SLICE)])
    plsc.subcore_barrier()

    pltpu.sync_copy(u_h.at[c], tbl_v)
    pltpu.sync_copy(zeros_h, acc_v)
    _local_edge_pass(e_h, sidx_v, didx_v, tbl_v, acc_v, crow, nch)
    pltpu.sync_copy(acc_v, vp_h.at[wid])


def _final_body(vp_ref, wp_ref, ns_ref, x_ref, w1_ref, b1_ref, w2_ref,
                b2_ref, out_ref):
    nsb = ns_ref[0]
    w = nsb * jnp.sum(wp_ref[...], axis=0)
    v = nsb * jnp.sum(vp_ref[...], axis=0)
    sw = jnp.sum(w)
    y = jnp.zeros((1, 128), F32)
    for k in range(ROWS):
        vrow = lax.slice(v, (k, 0), (k + 1, 128))
        xblk = x_ref[pl.ds(k * 128, 128), :]
        y = y + jnp.dot(vrow, xblk, preferred_element_type=F32)
    t = jnp.dot(y, w1_ref[...], preferred_element_type=F32)
    t = jnp.dot(t, w2_ref[...], preferred_element_type=F32)
    bias = sw * jnp.dot(b1_ref[...], w2_ref[...], preferred_element_type=F32)
    out_ref[...] = (t + bias) * (1.0 / N_NODES) + b2_ref[...]


_final_call = pl.pallas_call(
    _final_body,
    out_shape=jax.ShapeDtypeStruct((1, 64), F32),
)


def kernel(x, edge_index, W1, b1, W2, b2):
    e3 = edge_index.reshape(2, NCHUNKS, CHUNK)
    e_pad = jnp.pad(e3, ((0, 0), (0, NCHUNKS_PAD - NCHUNKS), (0, 0)))

    zeros = jnp.zeros((N_PAD,), F32)

    dop, dip = _deg_kernel(e_pad, zeros)
    wp, ns, nd = _w_kernel(e_pad, dop, dip, zeros)
    vp, _ = _v_kernel(e_pad, wp, ns, nd, zeros)

    xp = jnp.pad(x, ((0, N_PAD - N_NODES), (0, 0)))
    out = _final_call(
        vp.reshape(NUM_WORKERS, ROWS, 128),
        wp.reshape(NUM_WORKERS, ROWS, 128),
        ns.reshape(NUM_CORES, ROWS, 128), xp, W1, b1.reshape(1, -1), W2,
        b2.reshape(1, -1))
    return out


# speedup vs baseline: 1.2127x; 1.2127x over previous
import functools

import jax
import jax.numpy as jnp
from jax import lax
from jax.experimental import pallas as pl
from jax.experimental.pallas import tpu as pltpu
from jax.experimental.pallas import tpu_sc as plsc

N_NODES = 10000
N_PAD = 10240
E_EDGES = 320000
NUM_CORES = 2
NUM_SUBCORES = 16
CHUNK = 128
NCHUNKS = E_EDGES // CHUNK
NCHUNKS_PAD = 2560
CPW_MAX = 80
SLICE = N_PAD // NUM_SUBCORES
LANES = 16
F32 = jnp.float32

_MESH = plsc.VectorSubcoreMesh(core_axis_name="c", subcore_axis_name="s")


def _worker_chunks(wid):
    nch = jnp.where(wid < 31, 80, 20)
    crow = pl.multiple_of(wid * CPW_MAX, 8)
    return crow, nch


def _rsqrt16(d):
    s = jnp.full((LANES,), 1.0, F32)
    for _ in range(10):
        s = jnp.where(d * s * s >= 2.0, s * 0.5, s)
    y = s
    for _ in range(6):
        y = y * (1.5 - 0.5 * d * y * y)
    return y


@functools.partial(
    pl.kernel,
    out_type=(
        jax.ShapeDtypeStruct((NUM_CORES, N_PAD), F32),
        jax.ShapeDtypeStruct((NUM_CORES, N_PAD), F32),
    ),
    mesh=_MESH,
    scratch_types=(
        pltpu.VMEM((CPW_MAX, CHUNK), jnp.int32),
        pltpu.VMEM((CPW_MAX, CHUNK), jnp.int32),
        pltpu.VMEM((CHUNK,), F32),
        pltpu.VMEM((N_PAD,), F32),
        pltpu.VMEM_SHARED((N_PAD,), F32),
        pltpu.VMEM_SHARED((N_PAD,), F32),
        pltpu.SemaphoreType.DMA,
    ),
)
def _deg_kernel(e_h, zeros_h, ones_h, dop_h, dip_h, sidx_v, didx_v,
                ones_v, buf_v, acc_o, acc_i, ssem):
    c = lax.axis_index("c")
    s = lax.axis_index("s")
    wid = c * NUM_SUBCORES + s
    crow, nch = _worker_chunks(wid)

    pltpu.sync_copy(ones_h, ones_v)

    @pl.when(s == 0)
    def _():
        pltpu.sync_copy(zeros_h, buf_v)
        pltpu.sync_copy(buf_v, acc_o)
        pltpu.sync_copy(buf_v, acc_i)
    plsc.subcore_barrier()

    pltpu.sync_copy(e_h.at[0, pl.ds(crow, CPW_MAX)], sidx_v)
    pltpu.sync_copy(e_h.at[1, pl.ds(crow, CPW_MAX)], didx_v)

    def scat(j, _):
        pltpu.async_copy(ones_v, acc_o.at[sidx_v.at[j]], ssem, add=True)
        pltpu.async_copy(ones_v, acc_i.at[didx_v.at[j]], ssem, add=True)
        return 0
    lax.fori_loop(0, nch, scat, 0)

    def drain(j, _):
        pltpu.make_async_copy(ones_v, acc_o.at[sidx_v.at[0]], ssem).wait()
        return 0
    lax.fori_loop(0, 2 * nch, drain, 0)
    plsc.subcore_barrier()

    @pl.when(s == 0)
    def _():
        pltpu.sync_copy(acc_o, buf_v)
        pltpu.sync_copy(buf_v, dop_h.at[c])
        pltpu.sync_copy(acc_i, buf_v)
        pltpu.sync_copy(buf_v, dip_h.at[c])


def _edge_pass(e_h, sidx_v, didx_v, vals_v, tbl_s, acc_s, gsem, ssem,
               crow, nch):
    pltpu.sync_copy(e_h.at[0, pl.ds(crow, CPW_MAX)], sidx_v)
    pltpu.sync_copy(e_h.at[1, pl.ds(crow, CPW_MAX)], didx_v)

    def gath(j, _):
        pltpu.async_copy(tbl_s.at[didx_v.at[j]], vals_v.at[j], gsem)
        return 0
    lax.fori_loop(0, nch, gath, 0)

    def gdrain(j, _):
        pltpu.make_async_copy(tbl_s.at[didx_v.at[0]], vals_v.at[0], gsem).wait()
        return 0
    lax.fori_loop(0, nch, gdrain, 0)

    def scat(j, _):
        pltpu.async_copy(vals_v.at[j], acc_s.at[sidx_v.at[j]], ssem, add=True)
        return 0
    lax.fori_loop(0, nch, scat, 0)

    def sdrain(j, _):
        pltpu.make_async_copy(vals_v.at[0], acc_s.at[sidx_v.at[0]], ssem).wait()
        return 0
    lax.fori_loop(0, nch, sdrain, 0)


@functools.partial(
    pl.kernel,
    out_type=(
        jax.ShapeDtypeStruct((NUM_CORES, N_PAD), F32),
        jax.ShapeDtypeStruct((N_PAD,), F32),
        jax.ShapeDtypeStruct((N_PAD,), F32),
    ),
    mesh=_MESH,
    scratch_types=(
        pltpu.VMEM((CPW_MAX, CHUNK), jnp.int32),
        pltpu.VMEM((CPW_MAX, CHUNK), jnp.int32),
        pltpu.VMEM((CPW_MAX, CHUNK), F32),
        pltpu.VMEM((N_PAD,), F32),
        pltpu.VMEM((SLICE,), F32),
        pltpu.VMEM((SLICE,), F32),
        pltpu.VMEM((SLICE,), F32),
        pltpu.VMEM((SLICE,), F32),
        pltpu.VMEM_SHARED((N_PAD,), F32),
        pltpu.VMEM_SHARED((N_PAD,), F32),
        pltpu.SemaphoreType.DMA,
        pltpu.SemaphoreType.DMA,
    ),
)
def _w_kernel(e_h, dop_h, dip_h, zeros_h, wp_h, ns_h, nd_h,
              sidx_v, didx_v, vals_v, buf_v, t0, t1, ns_sl, nd_sl,
              tbl_s, acc_s, gsem, ssem):
    c = lax.axis_index("c")
    s = lax.axis_index("s")
    wid = c * NUM_SUBCORES + s
    crow, nch = _worker_chunks(wid)
    base = pl.multiple_of(s * SLICE, 8)

    @pl.when(s == 0)
    def _():
        pltpu.sync_copy(zeros_h, buf_v)
        pltpu.sync_copy(buf_v, acc_s)

    pltpu.sync_copy(dop_h.at[0, pl.ds(base, SLICE)], t0)
    pltpu.sync_copy(dop_h.at[1, pl.ds(base, SLICE)], t1)

    def mk_ns(i, _):
        sl = pl.ds(i * LANES, LANES)
        d = jnp.maximum(t0[sl] + t1[sl], 1.0)
        ns_sl[sl] = _rsqrt16(d)
        return 0
    lax.fori_loop(0, SLICE // LANES, mk_ns, 0)

    pltpu.sync_copy(dip_h.at[0, pl.ds(base, SLICE)], t0)
    pltpu.sync_copy(dip_h.at[1, pl.ds(base, SLICE)], t1)

    def mk_nd(i, _):
        sl = pl.ds(i * LANES, LANES)
        d = jnp.maximum(t0[sl] + t1[sl], 1.0)
        nd_sl[sl] = _rsqrt16(d)
        return 0
    lax.fori_loop(0, SLICE // LANES, mk_nd, 0)

    pltpu.sync_copy(nd_sl, tbl_s.at[pl.ds(base, SLICE)])

    @pl.when(c == 0)
    def _():
        pltpu.sync_copy(ns_sl, ns_h.at[pl.ds(base, SLICE)])
        pltpu.sync_copy(nd_sl, nd_h.at[pl.ds(base, SLICE)])
    plsc.subcore_barrier()

    _edge_pass(e_h, sidx_v, didx_v, vals_v, tbl_s, acc_s, gsem, ssem,
               crow, nch)
    plsc.subcore_barrier()

    @pl.when(s == 0)
    def _():
        pltpu.sync_copy(acc_s, buf_v)
        pltpu.sync_copy(buf_v, wp_h.at[c])


@functools.partial(
    pl.kernel,
    out_type=jax.ShapeDtypeStruct((NUM_CORES, N_PAD), F32),
    mesh=_MESH,
    scratch_types=(
        pltpu.VMEM((CPW_MAX, CHUNK), jnp.int32),
        pltpu.VMEM((CPW_MAX, CHUNK), jnp.int32),
        pltpu.VMEM((CPW_MAX, CHUNK), F32),
        pltpu.VMEM((N_PAD,), F32),
        pltpu.VMEM((SLICE,), F32),
        pltpu.VMEM((SLICE,), F32),
        pltpu.VMEM((SLICE,), F32),
        pltpu.VMEM((SLICE,), F32),
        pltpu.VMEM_SHARED((N_PAD,), F32),
        pltpu.VMEM_SHARED((N_PAD,), F32),
        pltpu.SemaphoreType.DMA,
        pltpu.SemaphoreType.DMA,
    ),
)
def _v_kernel(e_h, wp_h, ns_h, nd_h, zeros_h, vp_h,
              sidx_v, didx_v, vals_v, buf_v, t0, t1, ns_sl, nd_sl,
              tbl_s, acc_s, gsem, ssem):
    c = lax.axis_index("c")
    s = lax.axis_index("s")
    wid = c * NUM_SUBCORES + s
    crow, nch = _worker_chunks(wid)
    base = pl.multiple_of(s * SLICE, 8)

    @pl.when(s == 0)
    def _():
        pltpu.sync_copy(zeros_h, buf_v)
        pltpu.sync_copy(buf_v, acc_s)

    pltpu.sync_copy(wp_h.at[0, pl.ds(base, SLICE)], t0)
    pltpu.sync_copy(wp_h.at[1, pl.ds(base, SLICE)], t1)
    pltpu.sync_copy(ns_h.at[pl.ds(base, SLICE)], ns_sl)
    pltpu.sync_copy(nd_h.at[pl.ds(base, SLICE)], nd_sl)

    def mk_u(i, _):
        sl = pl.ds(i * LANES, LANES)
        t0[sl] = nd_sl[sl] * (ns_sl[sl] * (t0[sl] + t1[sl]))
        return 0
    lax.fori_loop(0, SLICE // LANES, mk_u, 0)

    pltpu.sync_copy(t0, tbl_s.at[pl.ds(base, SLICE)])
    plsc.subcore_barrier()

    _edge_pass(e_h, sidx_v, didx_v, vals_v, tbl_s, acc_s, gsem, ssem,
               crow, nch)
    plsc.subcore_barrier()

    @pl.when(s == 0)
    def _():
        pltpu.sync_copy(acc_s, buf_v)
        pltpu.sync_copy(buf_v, vp_h.at[c])


ROWS = N_PAD // 128


def _final_body(vp_ref, wp_ref, ns_ref, x_ref, w1_ref, b1_ref, w2_ref,
                b2_ref, out_ref):
    nsb = ns_ref[...]
    w = nsb * (wp_ref[0] + wp_ref[1])
    v = nsb * (vp_ref[0] + vp_ref[1])
    sw = jnp.sum(w)
    y = jnp.zeros((1, 128), F32)
    for k in range(ROWS):
        vrow = lax.slice(v, (k, 0), (k + 1, 128))
        xblk = x_ref[pl.ds(k * 128, 128), :]
        y = y + jnp.dot(vrow, xblk, preferred_element_type=F32)
    t = jnp.dot(y, w1_ref[...], preferred_element_type=F32)
    t = jnp.dot(t, w2_ref[...], preferred_element_type=F32)
    bias = sw * jnp.dot(b1_ref[...], w2_ref[...], preferred_element_type=F32)
    out_ref[...] = (t + bias) * (1.0 / N_NODES) + b2_ref[...]


_final_call = pl.pallas_call(
    _final_body,
    out_shape=jax.ShapeDtypeStruct((1, 64), F32),
)


def kernel(x, edge_index, W1, b1, W2, b2):
    e3 = edge_index.reshape(2, NCHUNKS, CHUNK)
    e_pad = jnp.pad(e3, ((0, 0), (0, NCHUNKS_PAD - NCHUNKS), (0, 0)))

    zeros = jnp.zeros((N_PAD,), F32)
    ones = jnp.ones((CHUNK,), F32)

    dop, dip = _deg_kernel(e_pad, zeros, ones)
    wp, ns, nd = _w_kernel(e_pad, dop, dip, zeros)
    vp = _v_kernel(e_pad, wp, ns, nd, zeros)

    xp = jnp.pad(x, ((0, N_PAD - N_NODES), (0, 0)))
    out = _final_call(
        vp.reshape(2, ROWS, 128), wp.reshape(2, ROWS, 128),
        ns.reshape(ROWS, 128), xp, W1, b1.reshape(1, -1), W2,
        b2.reshape(1, -1))
    return out
